# baseline (speedup 1.0000x reference)
import functools

import jax
import jax.numpy as jnp
from jax import lax
from jax.experimental import pallas as pl
from jax.experimental.pallas import tpu as pltpu
from jax.experimental.pallas import tpu_sc as plsc

N = 10000
E = 160000
D = 256
NC = 2
NS = 16
HALF = D // 2

K = 80
CHUNKS = E // K
CPT = CHUNKS // NS

KD = 40
DCHUNKS = E // KD
DCPT = DCHUNKS // (NC * NS)

NP = 10240
RPT = NP // NS

RB = 1280
GR = NP // RB

_mesh = plsc.VectorSubcoreMesh(core_axis_name="c", subcore_axis_name="s")
_sc_params = pltpu.CompilerParams(use_tc_tiling_on_sc=False)


@functools.partial(
    pl.kernel,
    mesh=_mesh,
    out_type=jax.ShapeDtypeStruct((NC, NS, RPT, 16), jnp.float32),
    scratch_types=[
        pltpu.VMEM((E // (NC * NS),), jnp.int32),
        pltpu.VMEM((KD, 16), jnp.float32),
        pltpu.VMEM_SHARED((NP, 16), jnp.float32),
        pltpu.SemaphoreType.DMA,
    ],
    compiler_params=_sc_params,
)
def _sc_degree(ei_hbm, ones_hbm, zeros_hbm, out_hbm, dbuf, ones_v,
               dacc, sem):
    c = lax.axis_index("c")
    s = lax.axis_index("s")
    w = c * NS + s
    ept = E // (NC * NS)

    pltpu.sync_copy(zeros_hbm, dacc.at[pl.ds(s * RPT, RPT)])
    pltpu.sync_copy(ones_hbm, ones_v)
    pltpu.sync_copy(ei_hbm.at[1].at[pl.ds(w * ept, ept)], dbuf)
    plsc.subcore_barrier()

    def body(i, carry):
        pltpu.sync_copy(ones_v, dacc.at[dbuf.at[pl.ds(i * KD, KD)]],
                        add=True)
        return carry

    lax.fori_loop(0, DCPT, body, 0)
    plsc.subcore_barrier()
    pltpu.sync_copy(dacc.at[pl.ds(s * RPT, RPT)], out_hbm.at[c, s])


@functools.partial(
    pl.kernel,
    mesh=_mesh,
    out_type=jax.ShapeDtypeStruct((NC, NS, RPT, HALF), jnp.float32),
    scratch_types=[
        pltpu.VMEM((E // NS,), jnp.int32),
        pltpu.VMEM((E // NS,), jnp.int32),
        pltpu.VMEM((K, HALF), jnp.float32),
        pltpu.VMEM((K, HALF), jnp.float32),
        pltpu.VMEM_SHARED((NP, HALF), jnp.float32),
        pltpu.SemaphoreType.DMA,
        pltpu.SemaphoreType.DMA,
    ],
    compiler_params=_sc_params,
)
def _sc_edge_pass(hp_hbm, ei_hbm, zeros_hbm, out_hbm, sbuf, dbuf,
                  r0, r1, acc, semg0, semg1):
    c = lax.axis_index("c")
    s = lax.axis_index("s")
    ept = E // NS

    pltpu.sync_copy(zeros_hbm, acc.at[pl.ds(s * RPT, RPT)])
    pltpu.sync_copy(ei_hbm.at[0].at[pl.ds(s * ept, ept)], sbuf)
    pltpu.sync_copy(ei_hbm.at[1].at[pl.ds(s * ept, ept)], dbuf)
    off = jnp.full((16,), c * NP, jnp.int32)

    def adj(i, carry):
        sl = pl.ds(i * 16, 16)
        sbuf[sl] = sbuf[sl] + off
        return carry

    lax.fori_loop(0, ept // 16, adj, 0)
    plsc.subcore_barrier()

    def gather(j, buf, sem):
        pltpu.make_async_copy(hp_hbm.at[sbuf.at[pl.ds(j * K, K)]], buf,
                              sem).start()

    def gwait(buf, sem):
        pltpu.make_async_copy(hp_hbm.at[sbuf.at[pl.ds(0, K)]], buf,
                              sem).wait()

    def scat(j, buf):
        pltpu.sync_copy(buf, acc.at[dbuf.at[pl.ds(j * K, K)]], add=True)

    gather(0, r0, semg0)
    gather(1, r1, semg1)

    def body(i, carry):
        j0 = 2 * i
        gwait(r0, semg0)
        scat(j0, r0)

        @pl.when(j0 + 2 < CPT)
        def _():
            gather(j0 + 2, r0, semg0)

        gwait(r1, semg1)
        scat(j0 + 1, r1)

        @pl.when(j0 + 3 < CPT)
        def _():
            gather(j0 + 3, r1, semg1)

        return carry

    lax.fori_loop(0, (CPT - 1) // 2, body, 0)
    gwait(r0, semg0)
    scat(CPT - 1, r0)

    plsc.subcore_barrier()
    pltpu.sync_copy(acc.at[pl.ds(s * RPT, RPT)], out_hbm.at[c, s])


def _prep_body(x_ref, w_ref, d0_ref, d1_ref, hp_ref, dinv_ref):
    deg0 = d0_ref[0][:, 0:1]
    deg1 = d1_ref[0][:, 0:1]
    dinv = lax.rsqrt(deg0 + deg1 + 1.0)
    dinv_ref[...] = dinv
    h = jnp.dot(x_ref[...], w_ref[...], preferred_element_type=jnp.float32)
    hp_ref[0] = dinv * h[:, :HALF]
    hp_ref[1] = dinv * h[:, HALF:]


def _tc_prep(x, w0, degs):
    return pl.pallas_call(
        _prep_body,
        grid=(GR,),
        in_specs=[
            pl.BlockSpec((RB, D), lambda i: (i, 0)),
            pl.BlockSpec((D, D), lambda i: (0, 0)),
            pl.BlockSpec((1, RB, 16), lambda i: (0, i, 0)),
            pl.BlockSpec((1, RB, 16), lambda i: (1, i, 0)),
        ],
        out_specs=[
            pl.BlockSpec((NC, RB, HALF), lambda i: (0, i, 0)),
            pl.BlockSpec((RB, 1), lambda i: (i, 0)),
        ],
        out_shape=[
            jax.ShapeDtypeStruct((NC, NP, HALF), jnp.float32),
            jax.ShapeDtypeStruct((NP, 1), jnp.float32),
        ],
    )(x, w0, degs, degs)


def _epilogue(acc0, acc1, hp0, hp1, dinv, b_ref, g_ref, h_ref):
    a0 = dinv * (acc0 + hp0) + b_ref[0:1, :]
    a1 = dinv * (acc1 + hp1) + b_ref[1:2, :]
    mu = (jnp.sum(a0, axis=1, keepdims=True)
          + jnp.sum(a1, axis=1, keepdims=True)) / D
    c0 = a0 - mu
    c1 = a1 - mu
    var = (jnp.sum(c0 * c0, axis=1, keepdims=True)
           + jnp.sum(c1 * c1, axis=1, keepdims=True)) / D
    inv = lax.rsqrt(var + 1e-5)
    y0 = jnp.maximum(g_ref[0:1, :] * (c0 * inv) + h_ref[0:1, :], 0.0)
    y1 = jnp.maximum(g_ref[1:2, :] * (c1 * inv) + h_ref[1:2, :], 0.0)
    return y0, y1


def _mid_body(a0_ref, a1_ref, p0_ref, p1_ref, dinv_ref, b_ref, g_ref, h_ref,
              w_ref, hp_ref):
    dinv = dinv_ref[...]
    y0, y1 = _epilogue(a0_ref[0], a1_ref[0], p0_ref[0], p1_ref[0],
                       dinv, b_ref, g_ref, h_ref)
    a = jnp.concatenate([y0, y1], axis=1)
    h = jnp.dot(a, w_ref[...], preferred_element_type=jnp.float32)
    hp_ref[0] = dinv * h[:, :HALF]
    hp_ref[1] = dinv * h[:, HALF:]


def _tc_mid(acc, hp, dinv, b, g, h, w_next):
    return pl.pallas_call(
        _mid_body,
        grid=(GR,),
        in_specs=[
            pl.BlockSpec((1, RB, HALF), lambda i: (0, i, 0)),
            pl.BlockSpec((1, RB, HALF), lambda i: (1, i, 0)),
            pl.BlockSpec((1, RB, HALF), lambda i: (0, i, 0)),
            pl.BlockSpec((1, RB, HALF), lambda i: (1, i, 0)),
            pl.BlockSpec((RB, 1), lambda i: (i, 0)),
            pl.BlockSpec((NC, HALF), lambda i: (0, 0)),
            pl.BlockSpec((NC, HALF), lambda i: (0, 0)),
            pl.BlockSpec((NC, HALF), lambda i: (0, 0)),
            pl.BlockSpec((D, D), lambda i: (0, 0)),
        ],
        out_specs=pl.BlockSpec((NC, RB, HALF), lambda i: (0, i, 0)),
        out_shape=jax.ShapeDtypeStruct((NC, NP, HALF), jnp.float32),
    )(acc, acc, hp, hp, dinv, b, g, h, w_next)


def _final_body(a0_ref, a1_ref, p0_ref, p1_ref, dinv_ref, b_ref, g_ref,
                h_ref, out_ref):
    y0, y1 = _epilogue(a0_ref[0], a1_ref[0], p0_ref[0], p1_ref[0],
                       dinv_ref[...], b_ref, g_ref, h_ref)
    out_ref[...] = jnp.concatenate([y0, y1], axis=1)


def _tc_final(acc, hp, dinv, b, g, h):
    return pl.pallas_call(
        _final_body,
        grid=(GR,),
        in_specs=[
            pl.BlockSpec((1, RB, HALF), lambda i: (0, i, 0)),
            pl.BlockSpec((1, RB, HALF), lambda i: (1, i, 0)),
            pl.BlockSpec((1, RB, HALF), lambda i: (0, i, 0)),
            pl.BlockSpec((1, RB, HALF), lambda i: (1, i, 0)),
            pl.BlockSpec((RB, 1), lambda i: (i, 0)),
            pl.BlockSpec((NC, HALF), lambda i: (0, 0)),
            pl.BlockSpec((NC, HALF), lambda i: (0, 0)),
            pl.BlockSpec((NC, HALF), lambda i: (0, 0)),
        ],
        out_specs=pl.BlockSpec((RB, D), lambda i: (i, 0)),
        out_shape=jax.ShapeDtypeStruct((N, D), jnp.float32),
    )(acc, acc, hp, hp, dinv, b, g, h)


def kernel(x, W0, b0, g0, h0, W1, b1, g1, h1, W2, b2, g2, h2, edge_index):
    ones16 = jnp.ones((KD, 16), jnp.float32)
    zeros16 = jnp.zeros((RPT, 16), jnp.float32)
    zeros128 = jnp.zeros((RPT, HALF), jnp.float32)

    degs = _sc_degree(edge_index, ones16, zeros16).reshape(NC, NP, 16)
    hp, dinv = _tc_prep(x, W0, degs)

    def edge(hp):
        return _sc_edge_pass(hp.reshape(NC * NP, HALF), edge_index,
                             zeros128).reshape(NC, NP, HALF)

    acc = edge(hp)
    hp = _tc_mid(acc, hp, dinv, b0.reshape(NC, HALF), g0.reshape(NC, HALF),
                 h0.reshape(NC, HALF), W1)

    acc = edge(hp)
    hp = _tc_mid(acc, hp, dinv, b1.reshape(NC, HALF), g1.reshape(NC, HALF),
                 h1.reshape(NC, HALF), W2)

    acc = edge(hp)
    return _tc_final(acc, hp, dinv, b2.reshape(NC, HALF),
                     g2.reshape(NC, HALF), h2.reshape(NC, HALF))


# speedup vs baseline: 1.0133x; 1.0133x over previous
import functools

import jax
import jax.numpy as jnp
from jax import lax
from jax.experimental import pallas as pl
from jax.experimental.pallas import tpu as pltpu
from jax.experimental.pallas import tpu_sc as plsc

N = 10000
E = 160000
D = 256
NC = 2
NS = 16
HALF = D // 2

K = 80
CHUNKS = E // K
CPT = CHUNKS // NS

KD = 40
DCHUNKS = E // KD
DCPT = DCHUNKS // (NC * NS)

NP = 10240
RPT = NP // NS

RB = 1280
GR = NP // RB

_mesh = plsc.VectorSubcoreMesh(core_axis_name="c", subcore_axis_name="s")
_sc_params = pltpu.CompilerParams(use_tc_tiling_on_sc=False)


@functools.partial(
    pl.kernel,
    mesh=_mesh,
    out_type=[
        jax.ShapeDtypeStruct((NC, NS, RPT, 16), jnp.float32),
        jax.ShapeDtypeStruct((E,), jnp.int32),
    ],
    scratch_types=[
        pltpu.VMEM((E // (NC * NS),), jnp.int32),
        pltpu.VMEM((E // (NC * NS) + 16,), jnp.int32),
        pltpu.VMEM((KD, 16), jnp.float32),
        pltpu.VMEM_SHARED((NP, 16), jnp.float32),
        pltpu.SemaphoreType.DMA,
    ],
    compiler_params=_sc_params,
)
def _sc_degree(ei_hbm, ones_hbm, zeros_hbm, out_hbm, srcadj_hbm, dbuf, sbuf,
               ones_v, dacc, sem):
    c = lax.axis_index("c")
    s = lax.axis_index("s")
    w = c * NS + s
    ept = E // (NC * NS)

    pltpu.sync_copy(zeros_hbm, dacc.at[pl.ds(s * RPT, RPT)])
    pltpu.sync_copy(ones_hbm, ones_v)
    pltpu.sync_copy(ei_hbm.at[1].at[pl.ds(w * ept, ept)], dbuf)
    pltpu.sync_copy(ei_hbm.at[0].at[pl.ds(w * ept, ept)],
                    sbuf.at[pl.ds(0, ept)])
    off = jnp.full((16,), NP, jnp.int32)

    def adj(i, carry):
        sl = pl.ds(i * 16, 16)
        sbuf[sl] = sbuf[sl] + off
        return carry

    lax.fori_loop(0, (ept + 15) // 16, adj, 0)
    pltpu.sync_copy(sbuf.at[pl.ds(0, ept)], srcadj_hbm.at[pl.ds(w * ept, ept)])
    plsc.subcore_barrier()

    def body(i, carry):
        pltpu.sync_copy(ones_v, dacc.at[dbuf.at[pl.ds(i * KD, KD)]],
                        add=True)
        return carry

    lax.fori_loop(0, DCPT, body, 0)
    plsc.subcore_barrier()
    pltpu.sync_copy(dacc.at[pl.ds(s * RPT, RPT)], out_hbm.at[c, s])


@functools.partial(
    pl.kernel,
    mesh=_mesh,
    out_type=jax.ShapeDtypeStruct((NC, NS, RPT, HALF), jnp.float32),
    scratch_types=[
        pltpu.VMEM((E // NS,), jnp.int32),
        pltpu.VMEM((E // NS,), jnp.int32),
        pltpu.VMEM((K, HALF), jnp.float32),
        pltpu.VMEM((K, HALF), jnp.float32),
        pltpu.VMEM_SHARED((NP, HALF), jnp.float32),
        pltpu.SemaphoreType.DMA,
        pltpu.SemaphoreType.DMA,
    ],
    compiler_params=_sc_params,
)
def _sc_edge_pass(hp_hbm, ei_hbm, srcadj_hbm, zeros_hbm, out_hbm, sbuf, dbuf,
                  r0, r1, acc, semg0, semg1):
    c = lax.axis_index("c")
    s = lax.axis_index("s")
    ept = E // NS

    pltpu.sync_copy(zeros_hbm, acc.at[pl.ds(s * RPT, RPT)])

    @pl.when(c == 0)
    def _():
        pltpu.sync_copy(ei_hbm.at[0].at[pl.ds(s * ept, ept)], sbuf)

    @pl.when(c == 1)
    def _():
        pltpu.sync_copy(srcadj_hbm.at[pl.ds(s * ept, ept)], sbuf)

    pltpu.sync_copy(ei_hbm.at[1].at[pl.ds(s * ept, ept)], dbuf)
    plsc.subcore_barrier()

    def gather(j, buf, sem):
        pltpu.make_async_copy(hp_hbm.at[sbuf.at[pl.ds(j * K, K)]], buf,
                              sem).start()

    def gwait(buf, sem):
        pltpu.make_async_copy(hp_hbm.at[sbuf.at[pl.ds(0, K)]], buf,
                              sem).wait()

    def scat(j, buf):
        pltpu.sync_copy(buf, acc.at[dbuf.at[pl.ds(j * K, K)]], add=True)

    gather(0, r0, semg0)
    gather(1, r1, semg1)

    def body(i, carry):
        j0 = 2 * i
        gwait(r0, semg0)
        scat(j0, r0)

        @pl.when(j0 + 2 < CPT)
        def _():
            gather(j0 + 2, r0, semg0)

        gwait(r1, semg1)
        scat(j0 + 1, r1)

        @pl.when(j0 + 3 < CPT)
        def _():
            gather(j0 + 3, r1, semg1)

        return carry

    lax.fori_loop(0, (CPT - 1) // 2, body, 0)
    gwait(r0, semg0)
    scat(CPT - 1, r0)

    plsc.subcore_barrier()
    pltpu.sync_copy(acc.at[pl.ds(s * RPT, RPT)], out_hbm.at[c, s])


def _prep_body(x_ref, w_ref, d0_ref, d1_ref, hp_ref, dinv_ref):
    deg0 = d0_ref[0][:, 0:1]
    deg1 = d1_ref[0][:, 0:1]
    dinv = lax.rsqrt(deg0 + deg1 + 1.0)
    dinv_ref[...] = dinv
    h = jnp.dot(x_ref[...], w_ref[...], preferred_element_type=jnp.float32)
    hp_ref[0] = dinv * h[:, :HALF]
    hp_ref[1] = dinv * h[:, HALF:]


def _tc_prep(x, w0, degs):
    return pl.pallas_call(
        _prep_body,
        grid=(GR,),
        in_specs=[
            pl.BlockSpec((RB, D), lambda i: (i, 0)),
            pl.BlockSpec((D, D), lambda i: (0, 0)),
            pl.BlockSpec((1, RB, 16), lambda i: (0, i, 0)),
            pl.BlockSpec((1, RB, 16), lambda i: (1, i, 0)),
        ],
        out_specs=[
            pl.BlockSpec((NC, RB, HALF), lambda i: (0, i, 0)),
            pl.BlockSpec((RB, 1), lambda i: (i, 0)),
        ],
        out_shape=[
            jax.ShapeDtypeStruct((NC, NP, HALF), jnp.float32),
            jax.ShapeDtypeStruct((NP, 1), jnp.float32),
        ],
    )(x, w0, degs, degs)


def _epilogue(acc0, acc1, hp0, hp1, dinv, b_ref, g_ref, h_ref):
    a0 = dinv * (acc0 + hp0) + b_ref[0:1, :]
    a1 = dinv * (acc1 + hp1) + b_ref[1:2, :]
    mu = (jnp.sum(a0, axis=1, keepdims=True)
          + jnp.sum(a1, axis=1, keepdims=True)) / D
    c0 = a0 - mu
    c1 = a1 - mu
    var = (jnp.sum(c0 * c0, axis=1, keepdims=True)
           + jnp.sum(c1 * c1, axis=1, keepdims=True)) / D
    inv = lax.rsqrt(var + 1e-5)
    y0 = jnp.maximum(g_ref[0:1, :] * (c0 * inv) + h_ref[0:1, :], 0.0)
    y1 = jnp.maximum(g_ref[1:2, :] * (c1 * inv) + h_ref[1:2, :], 0.0)
    return y0, y1


def _mid_body(a0_ref, a1_ref, p0_ref, p1_ref, dinv_ref, b_ref, g_ref, h_ref,
              w_ref, hp_ref):
    dinv = dinv_ref[...]
    y0, y1 = _epilogue(a0_ref[0], a1_ref[0], p0_ref[0], p1_ref[0],
                       dinv, b_ref, g_ref, h_ref)
    a = jnp.concatenate([y0, y1], axis=1)
    h = jnp.dot(a, w_ref[...], preferred_element_type=jnp.float32)
    hp_ref[0] = dinv * h[:, :HALF]
    hp_ref[1] = dinv * h[:, HALF:]


def _tc_mid(acc, hp, dinv, b, g, h, w_next):
    return pl.pallas_call(
        _mid_body,
        grid=(GR,),
        in_specs=[
            pl.BlockSpec((1, RB, HALF), lambda i: (0, i, 0)),
            pl.BlockSpec((1, RB, HALF), lambda i: (1, i, 0)),
            pl.BlockSpec((1, RB, HALF), lambda i: (0, i, 0)),
            pl.BlockSpec((1, RB, HALF), lambda i: (1, i, 0)),
            pl.BlockSpec((RB, 1), lambda i: (i, 0)),
            pl.BlockSpec((NC, HALF), lambda i: (0, 0)),
            pl.BlockSpec((NC, HALF), lambda i: (0, 0)),
            pl.BlockSpec((NC, HALF), lambda i: (0, 0)),
            pl.BlockSpec((D, D), lambda i: (0, 0)),
        ],
        out_specs=pl.BlockSpec((NC, RB, HALF), lambda i: (0, i, 0)),
        out_shape=jax.ShapeDtypeStruct((NC, NP, HALF), jnp.float32),
    )(acc, acc, hp, hp, dinv, b, g, h, w_next)


def _final_body(a0_ref, a1_ref, p0_ref, p1_ref, dinv_ref, b_ref, g_ref,
                h_ref, out_ref):
    y0, y1 = _epilogue(a0_ref[0], a1_ref[0], p0_ref[0], p1_ref[0],
                       dinv_ref[...], b_ref, g_ref, h_ref)
    out_ref[...] = jnp.concatenate([y0, y1], axis=1)


def _tc_final(acc, hp, dinv, b, g, h):
    return pl.pallas_call(
        _final_body,
        grid=(GR,),
        in_specs=[
            pl.BlockSpec((1, RB, HALF), lambda i: (0, i, 0)),
            pl.BlockSpec((1, RB, HALF), lambda i: (1, i, 0)),
            pl.BlockSpec((1, RB, HALF), lambda i: (0, i, 0)),
            pl.BlockSpec((1, RB, HALF), lambda i: (1, i, 0)),
            pl.BlockSpec((RB, 1), lambda i: (i, 0)),
            pl.BlockSpec((NC, HALF), lambda i: (0, 0)),
            pl.BlockSpec((NC, HALF), lambda i: (0, 0)),
            pl.BlockSpec((NC, HALF), lambda i: (0, 0)),
        ],
        out_specs=pl.BlockSpec((RB, D), lambda i: (i, 0)),
        out_shape=jax.ShapeDtypeStruct((N, D), jnp.float32),
    )(acc, acc, hp, hp, dinv, b, g, h)


def kernel(x, W0, b0, g0, h0, W1, b1, g1, h1, W2, b2, g2, h2, edge_index):
    ones16 = jnp.ones((KD, 16), jnp.float32)
    zeros16 = jnp.zeros((RPT, 16), jnp.float32)
    zeros128 = jnp.zeros((RPT, HALF), jnp.float32)

    degs, srcadj = _sc_degree(edge_index, ones16, zeros16)
    degs = degs.reshape(NC, NP, 16)
    hp, dinv = _tc_prep(x, W0, degs)

    def edge(hp):
        return _sc_edge_pass(hp.reshape(NC * NP, HALF), edge_index, srcadj,
                             zeros128).reshape(NC, NP, HALF)

    acc = edge(hp)
    hp = _tc_mid(acc, hp, dinv, b0.reshape(NC, HALF), g0.reshape(NC, HALF),
                 h0.reshape(NC, HALF), W1)

    acc = edge(hp)
    hp = _tc_mid(acc, hp, dinv, b1.reshape(NC, HALF), g1.reshape(NC, HALF),
                 h1.reshape(NC, HALF), W2)

    acc = edge(hp)
    return _tc_final(acc, hp, dinv, b2.reshape(NC, HALF),
                     g2.reshape(NC, HALF), h2.reshape(NC, HALF))


# speedup vs baseline: 1.0240x; 1.0106x over previous
import functools

import jax
import jax.numpy as jnp
from jax import lax
from jax.experimental import pallas as pl
from jax.experimental.pallas import tpu as pltpu
from jax.experimental.pallas import tpu_sc as plsc

N = 10000
E = 160000
D = 256
NC = 2
NS = 16
HALF = D // 2

K = 80
CHUNKS = E // K
CPT = CHUNKS // NS

KD = 40
DCHUNKS = E // KD
DCPT = DCHUNKS // (NC * NS)

NP = 10240
RPT = NP // NS

RB = 1280
GR = NP // RB

_mesh = plsc.VectorSubcoreMesh(core_axis_name="c", subcore_axis_name="s")
_sc_params = pltpu.CompilerParams(use_tc_tiling_on_sc=False)


@functools.partial(
    pl.kernel,
    mesh=_mesh,
    out_type=[
        jax.ShapeDtypeStruct((NC, NS, RPT, 16), jnp.float32),
        jax.ShapeDtypeStruct((E,), jnp.int32),
    ],
    scratch_types=[
        pltpu.VMEM((E // (NC * NS),), jnp.int32),
        pltpu.VMEM((E // (NC * NS) + 16,), jnp.int32),
        pltpu.VMEM((KD, 16), jnp.float32),
        pltpu.VMEM_SHARED((NP, 16), jnp.float32),
        pltpu.SemaphoreType.DMA,
    ],
    compiler_params=_sc_params,
)
def _sc_degree(ei_hbm, ones_hbm, zeros_hbm, out_hbm, srcadj_hbm, dbuf, sbuf,
               ones_v, dacc, sem):
    c = lax.axis_index("c")
    s = lax.axis_index("s")
    w = c * NS + s
    ept = E // (NC * NS)

    pltpu.sync_copy(zeros_hbm, dacc.at[pl.ds(s * RPT, RPT)])
    pltpu.sync_copy(ones_hbm, ones_v)
    pltpu.sync_copy(ei_hbm.at[1].at[pl.ds(w * ept, ept)], dbuf)
    pltpu.sync_copy(ei_hbm.at[0].at[pl.ds(w * ept, ept)],
                    sbuf.at[pl.ds(0, ept)])
    off = jnp.full((16,), NP, jnp.int32)

    def adj(i, carry):
        sl = pl.ds(i * 16, 16)
        sbuf[sl] = sbuf[sl] + off
        return carry

    lax.fori_loop(0, (ept + 15) // 16, adj, 0)
    pltpu.sync_copy(sbuf.at[pl.ds(0, ept)], srcadj_hbm.at[pl.ds(w * ept, ept)])
    plsc.subcore_barrier()

    def body(i, carry):
        pltpu.sync_copy(ones_v, dacc.at[dbuf.at[pl.ds(i * KD, KD)]],
                        add=True)
        return carry

    lax.fori_loop(0, DCPT, body, 0)
    plsc.subcore_barrier()
    pltpu.sync_copy(dacc.at[pl.ds(s * RPT, RPT)], out_hbm.at[c, s])


@functools.partial(
    pl.kernel,
    mesh=_mesh,
    out_type=jax.ShapeDtypeStruct((NC, NS, RPT, HALF), jnp.float32),
    scratch_types=[
        pltpu.VMEM((E // NS,), jnp.int32),
        pltpu.VMEM((E // NS,), jnp.int32),
        pltpu.VMEM((K, HALF), jnp.float32),
        pltpu.VMEM((K, HALF), jnp.float32),
        pltpu.VMEM_SHARED((NP, HALF), jnp.float32),
        pltpu.SemaphoreType.DMA,
        pltpu.SemaphoreType.DMA,
    ],
    compiler_params=_sc_params,
)
def _sc_edge_pass(hp_hbm, ei_hbm, srcadj_hbm, zeros_hbm, out_hbm, sbuf, dbuf,
                  r0, r1, acc, semg0, semg1):
    c = lax.axis_index("c")
    s = lax.axis_index("s")
    ept = E // NS

    zcp = pltpu.make_async_copy(zeros_hbm, acc.at[pl.ds(s * RPT, RPT)],
                                semg0)
    zcp.start()

    @pl.when(c == 0)
    def _():
        pltpu.sync_copy(ei_hbm.at[0].at[pl.ds(s * ept, ept)], sbuf)

    @pl.when(c == 1)
    def _():
        pltpu.sync_copy(srcadj_hbm.at[pl.ds(s * ept, ept)], sbuf)

    pltpu.sync_copy(ei_hbm.at[1].at[pl.ds(s * ept, ept)], dbuf)
    zcp.wait()
    plsc.subcore_barrier()

    def gather(j, buf, sem):
        pltpu.make_async_copy(hp_hbm.at[sbuf.at[pl.ds(j * K, K)]], buf,
                              sem).start()

    def gwait(buf, sem):
        pltpu.make_async_copy(hp_hbm.at[sbuf.at[pl.ds(0, K)]], buf,
                              sem).wait()

    def scat(j, buf):
        pltpu.sync_copy(buf, acc.at[dbuf.at[pl.ds(j * K, K)]], add=True)

    gather(0, r0, semg0)
    gather(1, r1, semg1)

    def body(i, carry):
        j0 = 2 * i
        gwait(r0, semg0)
        scat(j0, r0)

        @pl.when(j0 + 2 < CPT)
        def _():
            gather(j0 + 2, r0, semg0)

        gwait(r1, semg1)
        scat(j0 + 1, r1)

        @pl.when(j0 + 3 < CPT)
        def _():
            gather(j0 + 3, r1, semg1)

        return carry

    lax.fori_loop(0, (CPT - 1) // 2, body, 0)
    gwait(r0, semg0)
    scat(CPT - 1, r0)

    plsc.subcore_barrier()
    pltpu.sync_copy(acc.at[pl.ds(s * RPT, RPT)], out_hbm.at[c, s])


def _prep_body(x_ref, w_ref, d0_ref, d1_ref, hp_ref, dinv_ref):
    deg0 = d0_ref[0][:, 0:1]
    deg1 = d1_ref[0][:, 0:1]
    dinv = lax.rsqrt(deg0 + deg1 + 1.0)
    dinv_ref[...] = dinv
    h = jnp.dot(x_ref[...], w_ref[...], preferred_element_type=jnp.float32)
    hp_ref[0] = dinv * h[:, :HALF]
    hp_ref[1] = dinv * h[:, HALF:]


def _tc_prep(x, w0, degs):
    return pl.pallas_call(
        _prep_body,
        grid=(GR,),
        in_specs=[
            pl.BlockSpec((RB, D), lambda i: (i, 0)),
            pl.BlockSpec((D, D), lambda i: (0, 0)),
            pl.BlockSpec((1, RB, 16), lambda i: (0, i, 0)),
            pl.BlockSpec((1, RB, 16), lambda i: (1, i, 0)),
        ],
        out_specs=[
            pl.BlockSpec((NC, RB, HALF), lambda i: (0, i, 0)),
            pl.BlockSpec((RB, 1), lambda i: (i, 0)),
        ],
        out_shape=[
            jax.ShapeDtypeStruct((NC, NP, HALF), jnp.float32),
            jax.ShapeDtypeStruct((NP, 1), jnp.float32),
        ],
    )(x, w0, degs, degs)


def _epilogue(acc0, acc1, hp0, hp1, dinv, b_ref, g_ref, h_ref):
    a0 = dinv * (acc0 + hp0) + b_ref[0:1, :]
    a1 = dinv * (acc1 + hp1) + b_ref[1:2, :]
    mu = (jnp.sum(a0, axis=1, keepdims=True)
          + jnp.sum(a1, axis=1, keepdims=True)) / D
    c0 = a0 - mu
    c1 = a1 - mu
    var = (jnp.sum(c0 * c0, axis=1, keepdims=True)
           + jnp.sum(c1 * c1, axis=1, keepdims=True)) / D
    inv = lax.rsqrt(var + 1e-5)
    y0 = jnp.maximum(g_ref[0:1, :] * (c0 * inv) + h_ref[0:1, :], 0.0)
    y1 = jnp.maximum(g_ref[1:2, :] * (c1 * inv) + h_ref[1:2, :], 0.0)
    return y0, y1


def _mid_body(a0_ref, a1_ref, p0_ref, p1_ref, dinv_ref, b_ref, g_ref, h_ref,
              w_ref, hp_ref):
    dinv = dinv_ref[...]
    y0, y1 = _epilogue(a0_ref[0], a1_ref[0], p0_ref[0], p1_ref[0],
                       dinv, b_ref, g_ref, h_ref)
    a = jnp.concatenate([y0, y1], axis=1)
    h = jnp.dot(a, w_ref[...], preferred_element_type=jnp.float32)
    hp_ref[0] = dinv * h[:, :HALF]
    hp_ref[1] = dinv * h[:, HALF:]


def _tc_mid(acc, hp, dinv, b, g, h, w_next):
    return pl.pallas_call(
        _mid_body,
        grid=(GR,),
        in_specs=[
            pl.BlockSpec((1, RB, HALF), lambda i: (0, i, 0)),
            pl.BlockSpec((1, RB, HALF), lambda i: (1, i, 0)),
            pl.BlockSpec((1, RB, HALF), lambda i: (0, i, 0)),
            pl.BlockSpec((1, RB, HALF), lambda i: (1, i, 0)),
            pl.BlockSpec((RB, 1), lambda i: (i, 0)),
            pl.BlockSpec((NC, HALF), lambda i: (0, 0)),
            pl.BlockSpec((NC, HALF), lambda i: (0, 0)),
            pl.BlockSpec((NC, HALF), lambda i: (0, 0)),
            pl.BlockSpec((D, D), lambda i: (0, 0)),
        ],
        out_specs=pl.BlockSpec((NC, RB, HALF), lambda i: (0, i, 0)),
        out_shape=jax.ShapeDtypeStruct((NC, NP, HALF), jnp.float32),
    )(acc, acc, hp, hp, dinv, b, g, h, w_next)


def _final_body(a0_ref, a1_ref, p0_ref, p1_ref, dinv_ref, b_ref, g_ref,
                h_ref, out_ref):
    y0, y1 = _epilogue(a0_ref[0], a1_ref[0], p0_ref[0], p1_ref[0],
                       dinv_ref[...], b_ref, g_ref, h_ref)
    out_ref[...] = jnp.concatenate([y0, y1], axis=1)


def _tc_final(acc, hp, dinv, b, g, h):
    return pl.pallas_call(
        _final_body,
        grid=(GR,),
        in_specs=[
            pl.BlockSpec((1, RB, HALF), lambda i: (0, i, 0)),
            pl.BlockSpec((1, RB, HALF), lambda i: (1, i, 0)),
            pl.BlockSpec((1, RB, HALF), lambda i: (0, i, 0)),
            pl.BlockSpec((1, RB, HALF), lambda i: (1, i, 0)),
            pl.BlockSpec((RB, 1), lambda i: (i, 0)),
            pl.BlockSpec((NC, HALF), lambda i: (0, 0)),
            pl.BlockSpec((NC, HALF), lambda i: (0, 0)),
            pl.BlockSpec((NC, HALF), lambda i: (0, 0)),
        ],
        out_specs=pl.BlockSpec((RB, D), lambda i: (i, 0)),
        out_shape=jax.ShapeDtypeStruct((N, D), jnp.float32),
    )(acc, acc, hp, hp, dinv, b, g, h)


def kernel(x, W0, b0, g0, h0, W1, b1, g1, h1, W2, b2, g2, h2, edge_index):
    ones16 = jnp.ones((KD, 16), jnp.float32)
    zeros16 = jnp.zeros((RPT, 16), jnp.float32)
    zeros128 = jnp.zeros((RPT, HALF), jnp.float32)

    degs, srcadj = _sc_degree(edge_index, ones16, zeros16)
    degs = degs.reshape(NC, NP, 16)
    hp, dinv = _tc_prep(x, W0, degs)

    def edge(hp):
        return _sc_edge_pass(hp.reshape(NC * NP, HALF), edge_index, srcadj,
                             zeros128).reshape(NC, NP, HALF)

    acc = edge(hp)
    hp = _tc_mid(acc, hp, dinv, b0.reshape(NC, HALF), g0.reshape(NC, HALF),
                 h0.reshape(NC, HALF), W1)

    acc = edge(hp)
    hp = _tc_mid(acc, hp, dinv, b1.reshape(NC, HALF), g1.reshape(NC, HALF),
                 h1.reshape(NC, HALF), W2)

    acc = edge(hp)
    return _tc_final(acc, hp, dinv, b2.reshape(NC, HALF),
                     g2.reshape(NC, HALF), h2.reshape(NC, HALF))
